# baseline (speedup 1.0000x reference)
import jax
import jax.numpy as jnp
from jax import lax
from jax.experimental import pallas as pl
from jax.experimental.pallas import tpu as pltpu
from jax.experimental.pallas import tpu_sc as plsc

K = 1000000
D = 16
NTILES = 32
CHUNK = 2048
SC_ROWS = 999936
NFULL = SC_ROWS // CHUNK
HALF_START = NFULL * CHUNK
HALF_ROWS = SC_ROWS - HALF_START
HALF_TILE = NFULL % NTILES
TC_START = SC_ROWS


def _iota16():
    return lax.iota(jnp.int32, 16)


def _lane_gather(v, idx16):
    dn = lax.GatherDimensionNumbers(
        offset_dims=(), collapsed_slice_dims=(0,), start_index_map=(0,))
    return lax.gather(v, idx16[:, None], dn, slice_sizes=(1,),
                      mode=lax.GatherScatterMode.PROMISE_IN_BOUNDS)


def _splat(v, lane):
    return _lane_gather(v, jnp.full((16,), lane, jnp.int32))


def _merge_topk(cur_v, cur_i, blk_v, blk_i):
    sk, si = plsc.sort_key_val(blk_v, blk_i)
    rv = lax.rev(sk, (0,))
    ri = lax.rev(si, (0,))
    take_a = cur_v <= rv
    mv = jnp.where(take_a, cur_v, rv)
    mi = jnp.where(take_a, cur_i, ri)
    nv, ni = plsc.sort_key_val(mv, mi)
    return nv, ni, _splat(nv, 7)


def _sc_body(mnvT_hbm, smp_hbm, outv_hbm, outi_hbm,
             bufa, bufb, sbuf, resv, resi, sema, semb):
    cid = lax.axis_index("c")
    sid = lax.axis_index("s")
    wid = sid * 2 + cid

    pltpu.sync_copy(smp_hbm, sbuf)
    s = sbuf[...]
    iota = _iota16()
    inf = jnp.float32(jnp.inf)
    s_d = [_splat(s, d) for d in range(D)]

    def block_dist2(buf, b):
        lr0 = b * 16
        accs = [jnp.zeros((16,), jnp.float32) for _ in range(4)]
        for d in range(D):
            col = buf[d, pl.ds(lr0, 16)]
            diff = col - s_d[d]
            accs[d & 3] = accs[d & 3] + diff * diff
        return (accs[0] + accs[1]) + (accs[2] + accs[3])

    NB = 4

    def make_blk_body(buf, ok, row_start):
        def blk_body(bb, carry):
            cur_v, cur_i, tv = carry
            b = bb * NB
            accs = [block_dist2(buf, b + q) for q in range(NB)]
            gidx0 = row_start + b * 16 + iota
            lo = accs[0]
            for a in accs[1:]:
                lo = jnp.minimum(lo, a)
            hit = ok & jnp.any(lo < tv)

            def do(*args):
                gidx0 = args[NB]
                cur_v, cur_i, tv = args[NB + 1:]

                def mk(q):
                    def m(a, g, cur_v, cur_i, tv):
                        return _merge_topk(cur_v, cur_i, a, g)

                    def sk(a, g, cur_v, cur_i, tv):
                        return cur_v, cur_i, tv
                    return m, sk

                for q in range(NB):
                    a = args[q]
                    g = gidx0 + q * 16
                    m, sk = mk(q)
                    cur_v, cur_i, tv = lax.cond(
                        jnp.any(a < tv), m, sk, a, g, cur_v, cur_i, tv)
                return cur_v, cur_i, tv

            def skip(*args):
                return args[NB + 1:]

            return lax.cond(hit, do, skip,
                            *accs, gidx0, cur_v, cur_i, tv)
        return blk_body

    def slot_chunk(j):
        ch = wid + j * NTILES
        ok = ch < NFULL
        return jnp.where(ok, ch, wid), ok

    def dma(ch, buf, sem):
        return pltpu.make_async_copy(
            mnvT_hbm.at[:, pl.ds(ch * CHUNK, CHUNK)], buf, sem)

    def compute(ch, ok, buf, carry):
        return lax.fori_loop(
            0, CHUNK // (16 * 4), make_blk_body(buf, ok, ch * CHUNK), carry)

    NSLOT = (NFULL + NTILES - 1) // NTILES

    ch0, _ = slot_chunk(0)
    dma(ch0, bufa, sema).start()

    def pair_body(p, carry):
        j0 = p * 2
        ch0, ok0 = slot_chunk(j0)
        ch1, ok1 = slot_chunk(j0 + 1)
        ch2, _ = slot_chunk(j0 + 2)
        dma(ch0, bufa, sema).wait()
        dma(ch1, bufb, semb).start()
        carry = compute(ch0, ok0, bufa, carry)
        dma(ch1, bufb, semb).wait()
        dma(ch2, bufa, sema).start()
        return compute(ch1, ok1, bufb, carry)

    init = (jnp.full((16,), inf), jnp.zeros((16,), jnp.int32),
            jnp.full((16,), inf))
    cur_v, cur_i, tv = lax.fori_loop(0, NSLOT // 2, pair_body, init)
    chx, _ = slot_chunk(NSLOT)
    dma(chx, bufa, sema).wait()

    @pl.when(wid == HALF_TILE)
    def _():
        pltpu.sync_copy(
            mnvT_hbm.at[:, pl.ds(HALF_START, HALF_ROWS)],
            bufa.at[:, pl.ds(0, HALF_ROWS)])
        cv, ci, _t = lax.fori_loop(
            0, HALF_ROWS // (16 * 4),
            make_blk_body(bufa, True, HALF_START), (cur_v, cur_i, tv))
        resv[...] = cv
        resi[...] = ci

    @pl.when(wid != HALF_TILE)
    def _():
        resv[...] = cur_v
        resi[...] = cur_i

    pltpu.sync_copy(resv, outv_hbm.at[pl.ds(wid * 16, 16)])
    pltpu.sync_copy(resi, outi_hbm.at[pl.ds(wid * 16, 16)])


def _sc_dist_topk(mnvT, smp):
    mesh = plsc.VectorSubcoreMesh(core_axis_name="c", subcore_axis_name="s")
    f = pl.kernel(
        _sc_body,
        mesh=mesh,
        compiler_params=pltpu.CompilerParams(
            needs_layout_passes=False, use_tc_tiling_on_sc=True),
        out_type=[
            jax.ShapeDtypeStruct((NTILES * 16,), jnp.float32),
            jax.ShapeDtypeStruct((NTILES * 16,), jnp.int32),
        ],
        scratch_types=[
            pltpu.VMEM((D, CHUNK), jnp.float32),
            pltpu.VMEM((D, CHUNK), jnp.float32),
            pltpu.VMEM((16,), jnp.float32),
            pltpu.VMEM((16,), jnp.float32),
            pltpu.VMEM((16,), jnp.int32),
            pltpu.SemaphoreType.DMA,
            pltpu.SemaphoreType.DMA,
        ],
    )
    return f(mnvT, smp)


def _tc_merge_body(v_ref, i_ref, tail_ref, s_ref, idx_ref, val_ref):
    t = tail_ref[...]
    sv = s_ref[...]
    td = jnp.sum((t - sv) ** 2, axis=1)
    ti = TC_START + lax.iota(jnp.int32, 64)
    V = jnp.concatenate([v_ref[...], td])
    I = jnp.concatenate([i_ref[...], ti])
    big = jnp.int32(2**31 - 1)
    inf = jnp.float32(jnp.inf)
    idxs = []
    vals = []
    for _ in range(8):
        m = jnp.min(V)
        sel = V == m
        ci = jnp.min(jnp.where(sel, I, big))
        idxs.append(ci)
        vals.append(m)
        V = jnp.where(sel & (I == ci), inf, V)
    idx_ref[...] = jnp.stack(idxs)
    val_ref[...] = jnp.sqrt(jnp.stack(vals))


def _tc_merge(v2d, i2d, tail, smp):
    return pl.pallas_call(
        _tc_merge_body,
        out_shape=[
            jax.ShapeDtypeStruct((8,), jnp.int32),
            jax.ShapeDtypeStruct((8,), jnp.float32),
        ],
    )(v2d, i2d, tail, smp)


def kernel(samples, map_node_values, n):
    mnvT = map_node_values.T
    cv, ci = _sc_dist_topk(mnvT, samples)
    tail = map_node_values[TC_START:]
    idx, vals = _tc_merge(cv, ci, tail, samples.reshape(1, D))
    return idx, vals


# speedup vs baseline: 1.0859x; 1.0859x over previous
import jax
import jax.numpy as jnp
from jax import lax
from jax.experimental import pallas as pl
from jax.experimental.pallas import tpu as pltpu
from jax.experimental.pallas import tpu_sc as plsc

K = 1000000
D = 16
NTILES = 32
CHUNK = 2048
SC_ROWS = 999936
NFULL = SC_ROWS // CHUNK
HALF_START = NFULL * CHUNK
HALF_ROWS = SC_ROWS - HALF_START
HALF_TILE = NFULL % NTILES
TC_START = SC_ROWS


def _iota16():
    return lax.iota(jnp.int32, 16)


def _lane_gather(v, idx16):
    dn = lax.GatherDimensionNumbers(
        offset_dims=(), collapsed_slice_dims=(0,), start_index_map=(0,))
    return lax.gather(v, idx16[:, None], dn, slice_sizes=(1,),
                      mode=lax.GatherScatterMode.PROMISE_IN_BOUNDS)


def _splat(v, lane):
    return _lane_gather(v, jnp.full((16,), lane, jnp.int32))


def _merge_topk(cur_v, cur_i, blk_v, blk_i):
    sk, si = plsc.sort_key_val(blk_v, blk_i)
    rv = lax.rev(sk, (0,))
    ri = lax.rev(si, (0,))
    take_a = cur_v <= rv
    mv = jnp.where(take_a, cur_v, rv)
    mi = jnp.where(take_a, cur_i, ri)
    nv, ni = plsc.sort_key_val(mv, mi)
    return nv, ni, _splat(nv, 7)


def _sc_body(mnvT_hbm, smp_hbm, outv_hbm, outi_hbm,
             bufa, bufb, sbuf, resv, resi, sema, semb):
    cid = lax.axis_index("c")
    sid = lax.axis_index("s")
    wid = sid * 2 + cid

    pltpu.sync_copy(smp_hbm, sbuf)
    s = sbuf[...]
    iota = _iota16()
    inf = jnp.float32(jnp.inf)
    s_d = [_splat(s, d) for d in range(D)]

    def block_dist2(buf, b):
        lr0 = b * 16
        accs = [jnp.zeros((16,), jnp.float32) for _ in range(4)]
        for d in range(D):
            col = buf[d, pl.ds(lr0, 16)]
            diff = col - s_d[d]
            accs[d & 3] = accs[d & 3] + diff * diff
        return (accs[0] + accs[1]) + (accs[2] + accs[3])

    NB = 4

    def make_blk_body(buf, ok, row_start):
        def blk_body(bb, carry):
            cur_v, cur_i, tv = carry
            b = bb * NB
            accs = [block_dist2(buf, b + q) for q in range(NB)]
            gidx0 = row_start + b * 16 + iota
            lo = accs[0]
            for a in accs[1:]:
                lo = jnp.minimum(lo, a)
            hit = ok & jnp.any(lo < tv)

            def do(*args):
                gidx0 = args[NB]
                cur_v, cur_i, tv = args[NB + 1:]
                for q in range(NB):
                    cur_v, cur_i, tv = _merge_topk(
                        cur_v, cur_i, args[q], gidx0 + q * 16)
                return cur_v, cur_i, tv

            def skip(*args):
                return args[NB + 1:]

            return lax.cond(hit, do, skip,
                            *accs, gidx0, cur_v, cur_i, tv)
        return blk_body

    def slot_chunk(j):
        ch = wid + j * NTILES
        ok = ch < NFULL
        return jnp.where(ok, ch, wid), ok

    def dma(ch, buf, sem):
        return pltpu.make_async_copy(
            mnvT_hbm.at[:, pl.ds(ch * CHUNK, CHUNK)], buf, sem)

    def compute(ch, ok, buf, carry):
        return lax.fori_loop(
            0, CHUNK // (16 * 4), make_blk_body(buf, ok, ch * CHUNK), carry)

    NSLOT = (NFULL + NTILES - 1) // NTILES

    ch0, _ = slot_chunk(0)
    dma(ch0, bufa, sema).start()

    def pair_body(p, carry):
        j0 = p * 2
        ch0, ok0 = slot_chunk(j0)
        ch1, ok1 = slot_chunk(j0 + 1)
        ch2, _ = slot_chunk(j0 + 2)
        dma(ch0, bufa, sema).wait()
        dma(ch1, bufb, semb).start()
        carry = compute(ch0, ok0, bufa, carry)
        dma(ch1, bufb, semb).wait()
        dma(ch2, bufa, sema).start()
        return compute(ch1, ok1, bufb, carry)

    init = (jnp.full((16,), inf), jnp.zeros((16,), jnp.int32),
            jnp.full((16,), inf))
    cur_v, cur_i, tv = lax.fori_loop(0, NSLOT // 2, pair_body, init)
    chx, _ = slot_chunk(NSLOT)
    dma(chx, bufa, sema).wait()

    @pl.when(wid == HALF_TILE)
    def _():
        pltpu.sync_copy(
            mnvT_hbm.at[:, pl.ds(HALF_START, HALF_ROWS)],
            bufa.at[:, pl.ds(0, HALF_ROWS)])
        cv, ci, _t = lax.fori_loop(
            0, HALF_ROWS // (16 * 4),
            make_blk_body(bufa, True, HALF_START), (cur_v, cur_i, tv))
        resv[...] = cv
        resi[...] = ci

    @pl.when(wid != HALF_TILE)
    def _():
        resv[...] = cur_v
        resi[...] = cur_i

    pltpu.sync_copy(resv, outv_hbm.at[pl.ds(wid * 16, 16)])
    pltpu.sync_copy(resi, outi_hbm.at[pl.ds(wid * 16, 16)])


def _sc_dist_topk(mnvT, smp):
    mesh = plsc.VectorSubcoreMesh(core_axis_name="c", subcore_axis_name="s")
    f = pl.kernel(
        _sc_body,
        mesh=mesh,
        compiler_params=pltpu.CompilerParams(
            needs_layout_passes=False, use_tc_tiling_on_sc=True),
        out_type=[
            jax.ShapeDtypeStruct((NTILES * 16,), jnp.float32),
            jax.ShapeDtypeStruct((NTILES * 16,), jnp.int32),
        ],
        scratch_types=[
            pltpu.VMEM((D, CHUNK), jnp.float32),
            pltpu.VMEM((D, CHUNK), jnp.float32),
            pltpu.VMEM((16,), jnp.float32),
            pltpu.VMEM((16,), jnp.float32),
            pltpu.VMEM((16,), jnp.int32),
            pltpu.SemaphoreType.DMA,
            pltpu.SemaphoreType.DMA,
        ],
    )
    return f(mnvT, smp)


def _tc_merge_body(v_ref, i_ref, tail_ref, s_ref, idx_ref, val_ref):
    t = tail_ref[...]
    sv = s_ref[...]
    td = jnp.sum((t - sv) ** 2, axis=1)
    ti = TC_START + lax.iota(jnp.int32, 64)
    V = jnp.concatenate([v_ref[...], td])
    I = jnp.concatenate([i_ref[...], ti])
    big = jnp.int32(2**31 - 1)
    inf = jnp.float32(jnp.inf)
    idxs = []
    vals = []
    for _ in range(8):
        m = jnp.min(V)
        sel = V == m
        ci = jnp.min(jnp.where(sel, I, big))
        idxs.append(ci)
        vals.append(m)
        V = jnp.where(sel & (I == ci), inf, V)
    idx_ref[...] = jnp.stack(idxs)
    val_ref[...] = jnp.sqrt(jnp.stack(vals))


def _tc_merge(v2d, i2d, tail, smp):
    return pl.pallas_call(
        _tc_merge_body,
        out_shape=[
            jax.ShapeDtypeStruct((8,), jnp.int32),
            jax.ShapeDtypeStruct((8,), jnp.float32),
        ],
    )(v2d, i2d, tail, smp)


def kernel(samples, map_node_values, n):
    mnvT = map_node_values.T
    cv, ci = _sc_dist_topk(mnvT, samples)
    tail = map_node_values[TC_START:]
    idx, vals = _tc_merge(cv, ci, tail, samples.reshape(1, D))
    return idx, vals


# speedup vs baseline: 1.0884x; 1.0024x over previous
import jax
import jax.numpy as jnp
from jax import lax
from jax.experimental import pallas as pl
from jax.experimental.pallas import tpu as pltpu
from jax.experimental.pallas import tpu_sc as plsc

K = 1000000
D = 16
NTILES = 32
CHUNK = 2048
SC_ROWS = 999936
NFULL = SC_ROWS // CHUNK
HALF_START = NFULL * CHUNK
HALF_ROWS = SC_ROWS - HALF_START
HALF_TILE = NFULL % NTILES
TC_START = SC_ROWS


def _iota16():
    return lax.iota(jnp.int32, 16)


def _lane_gather(v, idx16):
    dn = lax.GatherDimensionNumbers(
        offset_dims=(), collapsed_slice_dims=(0,), start_index_map=(0,))
    return lax.gather(v, idx16[:, None], dn, slice_sizes=(1,),
                      mode=lax.GatherScatterMode.PROMISE_IN_BOUNDS)


def _splat(v, lane):
    return _lane_gather(v, jnp.full((16,), lane, jnp.int32))


def _merge_topk(cur_v, cur_i, blk_v, blk_i):
    sk, si = plsc.sort_key_val(blk_v, blk_i)
    rv = lax.rev(sk, (0,))
    ri = lax.rev(si, (0,))
    take_a = cur_v <= rv
    mv = jnp.where(take_a, cur_v, rv)
    mi = jnp.where(take_a, cur_i, ri)
    nv, ni = plsc.sort_key_val(mv, mi)
    return nv, ni, _splat(nv, 7)


def _sc_body(mnvT_hbm, smp_hbm, outv_hbm, outi_hbm,
             bufa, bufb, sbuf, resv, resi, sema, semb):
    cid = lax.axis_index("c")
    sid = lax.axis_index("s")
    wid = sid * 2 + cid

    pltpu.sync_copy(smp_hbm, sbuf)
    s = sbuf[...]
    iota = _iota16()
    inf = jnp.float32(jnp.inf)
    s_d = [_splat(s, d) for d in range(D)]

    def block_dist2(buf, b):
        lr0 = b * 16
        accs = [jnp.zeros((16,), jnp.float32) for _ in range(4)]
        for d in range(D):
            col = buf[d, pl.ds(lr0, 16)]
            diff = col - s_d[d]
            accs[d & 3] = accs[d & 3] + diff * diff
        return (accs[0] + accs[1]) + (accs[2] + accs[3])

    NB = 4

    def make_blk_body(buf, ok, row_start):
        def blk_body(bb, carry):
            cur_v, cur_i, tv = carry
            b = bb * NB
            accs = [block_dist2(buf, b + q) for q in range(NB)]
            gidx0 = row_start + b * 16 + iota
            lo = accs[0]
            for a in accs[1:]:
                lo = jnp.minimum(lo, a)
            hit = ok & jnp.any(lo < tv)

            def do(*args):
                gidx0 = args[NB]
                cur_v, cur_i, tv = args[NB + 1:]
                for q in range(NB):
                    cur_v, cur_i, tv = _merge_topk(
                        cur_v, cur_i, args[q], gidx0 + q * 16)
                return cur_v, cur_i, tv

            def skip(*args):
                return args[NB + 1:]

            return lax.cond(hit, do, skip,
                            *accs, gidx0, cur_v, cur_i, tv)
        return blk_body

    def slot_chunk(j):
        ch = wid + j * NTILES
        ok = ch < NFULL
        return jnp.where(ok, ch, wid), ok

    def dma(ch, buf, sem):
        return pltpu.make_async_copy(
            mnvT_hbm.at[:, pl.ds(ch * CHUNK, CHUNK)], buf, sem)

    def compute(ch, ok, buf, carry):
        return lax.fori_loop(
            0, CHUNK // (16 * 4), make_blk_body(buf, ok, ch * CHUNK), carry)

    NSLOT = (NFULL + NTILES - 1) // NTILES

    ch0, _ = slot_chunk(0)
    dma(ch0, bufa, sema).start()

    def pair_body(p, carry):
        j0 = p * 2
        ch0, ok0 = slot_chunk(j0)
        ch1, ok1 = slot_chunk(j0 + 1)
        ch2, _ = slot_chunk(j0 + 2)
        dma(ch0, bufa, sema).wait()
        dma(ch1, bufb, semb).start()
        carry = compute(ch0, ok0, bufa, carry)
        dma(ch1, bufb, semb).wait()
        dma(ch2, bufa, sema).start()
        return compute(ch1, ok1, bufb, carry)

    init = (jnp.full((16,), inf), jnp.zeros((16,), jnp.int32),
            jnp.full((16,), inf))
    cur_v, cur_i, tv = lax.fori_loop(0, NSLOT // 2, pair_body, init)
    chx, _ = slot_chunk(NSLOT)
    dma(chx, bufa, sema).wait()

    @pl.when(wid == HALF_TILE)
    def _():
        pltpu.sync_copy(
            mnvT_hbm.at[:, pl.ds(HALF_START, HALF_ROWS)],
            bufa.at[:, pl.ds(0, HALF_ROWS)])
        cv, ci, _t = lax.fori_loop(
            0, HALF_ROWS // (16 * 4),
            make_blk_body(bufa, True, HALF_START), (cur_v, cur_i, tv))
        resv[...] = cv
        resi[...] = ci

    @pl.when(wid != HALF_TILE)
    def _():
        resv[...] = cur_v
        resi[...] = cur_i

    pltpu.sync_copy(resv, outv_hbm.at[pl.ds(wid * 16, 16)])
    pltpu.sync_copy(resi, outi_hbm.at[pl.ds(wid * 16, 16)])


def _sc_dist_topk(mnvT, smp):
    mesh = plsc.VectorSubcoreMesh(core_axis_name="c", subcore_axis_name="s")
    f = pl.kernel(
        _sc_body,
        mesh=mesh,
        compiler_params=pltpu.CompilerParams(
            needs_layout_passes=False, use_tc_tiling_on_sc=True),
        out_type=[
            jax.ShapeDtypeStruct((NTILES * 16,), jnp.float32),
            jax.ShapeDtypeStruct((NTILES * 16,), jnp.int32),
        ],
        scratch_types=[
            pltpu.VMEM((D, CHUNK), jnp.float32),
            pltpu.VMEM((D, CHUNK), jnp.float32),
            pltpu.VMEM((16,), jnp.float32),
            pltpu.VMEM((16,), jnp.float32),
            pltpu.VMEM((16,), jnp.int32),
            pltpu.SemaphoreType.DMA,
            pltpu.SemaphoreType.DMA,
        ],
    )
    return f(mnvT, smp)


def _tc_merge_body(v_ref, i_ref, tail_ref, s_ref, idx_ref, val_ref):
    t = tail_ref[...]
    sv = s_ref[...]
    td = jnp.sum((t - sv) ** 2, axis=0)
    td = jnp.where(lax.iota(jnp.int32, 128) < K - TC_START, td,
                   jnp.float32(jnp.inf))
    ti = TC_START + lax.iota(jnp.int32, 128)
    V = jnp.concatenate([v_ref[...], td])
    I = jnp.concatenate([i_ref[...], ti])
    big = jnp.int32(2**31 - 1)
    inf = jnp.float32(jnp.inf)
    idxs = []
    vals = []
    for _ in range(8):
        m = jnp.min(V)
        sel = V == m
        ci = jnp.min(jnp.where(sel, I, big))
        idxs.append(ci)
        vals.append(m)
        V = jnp.where(sel & (I == ci), inf, V)
    idx_ref[...] = jnp.stack(idxs)
    val_ref[...] = jnp.sqrt(jnp.stack(vals))


def _tc_merge(cv, ci, mnvT, smp):
    return pl.pallas_call(
        _tc_merge_body,
        grid=(1,),
        in_specs=[
            pl.BlockSpec((NTILES * 16,), lambda i: (0,)),
            pl.BlockSpec((NTILES * 16,), lambda i: (0,)),
            pl.BlockSpec((D, 128), lambda i: (0, TC_START // 128)),
            pl.BlockSpec((D, 1), lambda i: (0, 0)),
        ],
        out_specs=[
            pl.BlockSpec((8,), lambda i: (0,)),
            pl.BlockSpec((8,), lambda i: (0,)),
        ],
        out_shape=[
            jax.ShapeDtypeStruct((8,), jnp.int32),
            jax.ShapeDtypeStruct((8,), jnp.float32),
        ],
    )(cv, ci, mnvT, smp)


def kernel(samples, map_node_values, n):
    mnvT = map_node_values.T
    cv, ci = _sc_dist_topk(mnvT, samples)
    idx, vals = _tc_merge(cv, ci, mnvT, samples.reshape(D, 1))
    return idx, vals
